# baseline (speedup 1.0000x reference)
import functools
import math

import jax
import jax.numpy as jnp
from jax import lax
from jax.experimental import pallas as pl
from jax.experimental.pallas import tpu as pltpu
from jax.experimental.pallas import tpu_sc as plsc

_B, _D, _CV, _CA, _S = 1024, 512, 2048, 2048, 32
_LR = 0.1
_BM = 256
_NC, _NS, _L = 2, 16, 16
_NW = _NC * _NS
_RPW = _B // _NW


def _topk_mask(scores, k):
    def keymap(x):
        u = jax.lax.bitcast_convert_type(x, jnp.uint32)
        return jnp.where((u >> jnp.uint32(31)) != jnp.uint32(0),
                         ~u, u | jnp.uint32(0x80000000))

    key = keymap(scores)
    key_t = keymap(jnp.transpose(scores))
    thr_row = jnp.zeros((1, scores.shape[0]), jnp.uint32)
    for bit in range(31, -1, -1):
        cand = thr_row | jnp.uint32(1 << bit)
        cnt = jnp.sum((key_t >= cand).astype(jnp.int32), axis=0,
                      keepdims=True)
        thr_row = jnp.where(cnt >= k, cand, thr_row)
    thr = jnp.transpose(thr_row)
    return jnp.where(key >= thr, jnp.float32(1.0), jnp.float32(0.0))


def _encode_body(keys_ref, pv_ref, mv_ref, tg_ref, a1_ref, a1f_ref, d_ref):
    s = jax.lax.dot_general(keys_ref[...], pv_ref[...],
                            (((1,), (1,)), ((), ())),
                            preferred_element_type=jnp.float32)
    m = _topk_mask(s, _S)
    a1f_ref[...] = m
    a1_ref[...] = m.astype(jnp.bfloat16)
    retrieved = jax.lax.dot_general(m, mv_ref[...], (((1,), (0,)), ((), ())),
                                    precision=jax.lax.Precision.HIGHEST,
                                    preferred_element_type=jnp.float32)
    deltas = (tg_ref[...] - retrieved) / _S * _LR
    d_ref[...] = jnp.broadcast_to(deltas, (deltas.shape[0], _L))


def _assoc_encode_body(a1_ref, ce_ref, pa_ref, a2_ref):
    a1f = a1_ref[...].astype(jnp.float32)
    dn = (((1,), (0,)), ((), ()))
    p = jax.lax.dot_general(a1f, ce_ref[...], dn,
                            precision=jax.lax.Precision.HIGHEST,
                            preferred_element_type=jnp.float32)
    p = p / jnp.float32(math.sqrt(_S))
    nrm = jnp.sqrt(jnp.sum(p * p, axis=1, keepdims=True))
    p = p / jnp.maximum(nrm, jnp.float32(1e-12))
    s2 = jax.lax.dot_general(p, pa_ref[...], (((1,), (1,)), ((), ())),
                             preferred_element_type=jnp.float32)
    a2_ref[...] = _topk_mask(s2, _S).astype(jnp.bfloat16)


def _val_sc_body(a1f_hbm, d_hbm, out_hbm, mrows, drows, acc_v):
    wid = lax.axis_index("s") * _NC + lax.axis_index("c")
    base = wid * _RPW
    pltpu.sync_copy(a1f_hbm.at[pl.ds(base, _RPW)], mrows)
    pltpu.sync_copy(d_hbm.at[pl.ds(base, _RPW)], drows)
    zeros16 = jnp.zeros((_L,), jnp.float32)
    for seg in range(_CV // _L):
        acc_v[pl.ds(seg * _L, _L)] = zeros16

    def row_body(r, carry):
        dvec = drows[r, :]

        def seg_add(seg, c):
            v = mrows[r, pl.ds(seg * _L, _L)]
            acc_v[pl.ds(seg * _L, _L)] += jnp.where(v > 0.0, dvec, 0.0)
            return c

        lax.fori_loop(0, _CV // _L, seg_add, 0)
        return carry

    lax.fori_loop(0, _RPW, row_body, 0)
    pltpu.sync_copy(acc_v, out_hbm.at[wid])


_val_sc = functools.partial(
    pl.kernel,
    mesh=plsc.VectorSubcoreMesh(core_axis_name="c", subcore_axis_name="s"),
    out_type=jax.ShapeDtypeStruct((_NW, _CV), jnp.float32),
    scratch_types=[
        pltpu.VMEM((_RPW, _CV), jnp.float32),
        pltpu.VMEM((_RPW, _L), jnp.float32),
        pltpu.VMEM((_CV,), jnp.float32),
    ],
)(_val_sc_body)


def _assoc_update_body(a2_ref, a1_ref, ma_ref, vp_ref, mv_ref, out_ref):
    scale = (jnp.float32(1.0) / jnp.float32(_S)) * jnp.float32(_LR)
    upd = jax.lax.dot_general(a2_ref[...], a1_ref[...],
                              (((0,), (0,)), ((), ())),
                              preferred_element_type=jnp.float32)
    ones = jnp.ones((_NW, 1), jnp.float32)
    val = mv_ref[...] + jax.lax.dot_general(
        vp_ref[...], ones, (((0,), (0,)), ((), ())),
        precision=jax.lax.Precision.HIGHEST,
        preferred_element_type=jnp.float32)
    out_ref[...] = jnp.concatenate([val, ma_ref[...] + upd * scale], axis=1)


def kernel(keys, targets, proj_value, clique_encoder, proj_assoc,
           mem_value_val, mem_value_assoc):
    nb = _B // _BM
    a1, a1f, dbc = pl.pallas_call(
        _encode_body,
        grid=(nb,),
        in_specs=[
            pl.BlockSpec((_BM, _D), lambda i: (i, 0)),
            pl.BlockSpec((_CV, _D), lambda i: (0, 0)),
            pl.BlockSpec((_CV, 1), lambda i: (0, 0)),
            pl.BlockSpec((_BM, 1), lambda i: (i, 0)),
        ],
        out_specs=[
            pl.BlockSpec((_BM, _CV), lambda i: (i, 0)),
            pl.BlockSpec((_BM, _CV), lambda i: (i, 0)),
            pl.BlockSpec((_BM, _L), lambda i: (i, 0)),
        ],
        out_shape=[
            jax.ShapeDtypeStruct((_B, _CV), jnp.bfloat16),
            jax.ShapeDtypeStruct((_B, _CV), jnp.float32),
            jax.ShapeDtypeStruct((_B, _L), jnp.float32),
        ],
    )(keys, proj_value, mem_value_val, targets)

    val_partials = _val_sc(a1f, dbc)

    a2 = pl.pallas_call(
        _assoc_encode_body,
        grid=(nb,),
        in_specs=[
            pl.BlockSpec((_BM, _CV), lambda i: (i, 0)),
            pl.BlockSpec((_CV, _CA), lambda i: (0, 0)),
            pl.BlockSpec((_CA, _CA), lambda i: (0, 0)),
        ],
        out_specs=pl.BlockSpec((_BM, _CA), lambda i: (i, 0)),
        out_shape=jax.ShapeDtypeStruct((_B, _CA), jnp.bfloat16),
    )(a1, clique_encoder, proj_assoc)

    bn = 256
    out = pl.pallas_call(
        _assoc_update_body,
        grid=(_CA // bn,),
        in_specs=[
            pl.BlockSpec((_B, bn), lambda j: (0, j)),
            pl.BlockSpec((_B, _CV), lambda j: (0, 0)),
            pl.BlockSpec((bn, _CV), lambda j: (j, 0)),
            pl.BlockSpec((_NW, bn), lambda j: (0, j)),
            pl.BlockSpec((bn, 1), lambda j: (j, 0)),
        ],
        out_specs=pl.BlockSpec((bn, 1 + _CV), lambda j: (j, 0)),
        out_shape=jax.ShapeDtypeStruct((_CA, 1 + _CV), jnp.float32),
    )(a2, a1, mem_value_assoc, val_partials, mem_value_val)

    return out


# speedup vs baseline: 1.1484x; 1.1484x over previous
import functools
import math

import jax
import jax.numpy as jnp
from jax import lax
from jax.experimental import pallas as pl
from jax.experimental.pallas import tpu as pltpu
from jax.experimental.pallas import tpu_sc as plsc

_B, _D, _CV, _CA, _S = 1024, 512, 2048, 2048, 32
_LR = 0.1
_BM = 256
_NC, _NS, _L = 2, 16, 16
_NW = _NC * _NS
_RPW = _B // _NW


def _topk_mask(scores, k):
    u = jax.lax.bitcast_convert_type(scores, jnp.uint32)
    key = jnp.where((u >> jnp.uint32(31)) != jnp.uint32(0),
                    ~u, u | jnp.uint32(0x80000000))
    thr = jnp.zeros((scores.shape[0], 1), jnp.uint32)
    for bit in range(31, -1, -1):
        cand = thr | jnp.uint32(1 << bit)
        cnt = jnp.sum((key >= cand).astype(jnp.int32), axis=1, keepdims=True)
        thr = jnp.where(cnt >= k, cand, thr)
    return jnp.where(key >= thr, jnp.float32(1.0), jnp.float32(0.0))


def _encode_body(keys_ref, pv_ref, mv_ref, tg_ref, a1_ref, a1f_ref, d_ref):
    s = jax.lax.dot_general(keys_ref[...], pv_ref[...],
                            (((1,), (1,)), ((), ())),
                            preferred_element_type=jnp.float32)
    m = _topk_mask(s, _S)
    a1f_ref[...] = m
    a1_ref[...] = m.astype(jnp.bfloat16)
    retrieved = jax.lax.dot_general(m, mv_ref[...], (((1,), (0,)), ((), ())),
                                    precision=jax.lax.Precision.HIGHEST,
                                    preferred_element_type=jnp.float32)
    deltas = (tg_ref[...] - retrieved) / _S * _LR
    d_ref[...] = jnp.broadcast_to(deltas, (deltas.shape[0], _L))


def _assoc_encode_body(a1_ref, ce_ref, pa_ref, a2_ref):
    a1f = a1_ref[...].astype(jnp.float32)
    dn = (((1,), (0,)), ((), ()))
    p = jax.lax.dot_general(a1f, ce_ref[...], dn,
                            precision=jax.lax.Precision.HIGHEST,
                            preferred_element_type=jnp.float32)
    p = p / jnp.float32(math.sqrt(_S))
    nrm = jnp.sqrt(jnp.sum(p * p, axis=1, keepdims=True))
    p = p / jnp.maximum(nrm, jnp.float32(1e-12))
    s2 = jax.lax.dot_general(p, pa_ref[...], (((1,), (1,)), ((), ())),
                             preferred_element_type=jnp.float32)
    a2_ref[...] = _topk_mask(s2, _S).astype(jnp.bfloat16)


def _val_sc_body(a1f_hbm, d_hbm, out_hbm, mrows, drows, acc_v):
    wid = lax.axis_index("s") * _NC + lax.axis_index("c")
    base = wid * _RPW
    pltpu.sync_copy(a1f_hbm.at[pl.ds(base, _RPW)], mrows)
    pltpu.sync_copy(d_hbm.at[pl.ds(base, _RPW)], drows)
    zeros16 = jnp.zeros((_L,), jnp.float32)
    for seg in range(_CV // _L):
        acc_v[pl.ds(seg * _L, _L)] = zeros16

    def row_body(r, carry):
        dvec = drows[r, :]

        def seg_add(seg, c):
            v = mrows[r, pl.ds(seg * _L, _L)]
            acc_v[pl.ds(seg * _L, _L)] += jnp.where(v > 0.0, dvec, 0.0)
            return c

        lax.fori_loop(0, _CV // _L, seg_add, 0)
        return carry

    lax.fori_loop(0, _RPW, row_body, 0)
    pltpu.sync_copy(acc_v, out_hbm.at[wid])


_val_sc = functools.partial(
    pl.kernel,
    mesh=plsc.VectorSubcoreMesh(core_axis_name="c", subcore_axis_name="s"),
    out_type=jax.ShapeDtypeStruct((_NW, _CV), jnp.float32),
    scratch_types=[
        pltpu.VMEM((_RPW, _CV), jnp.float32),
        pltpu.VMEM((_RPW, _L), jnp.float32),
        pltpu.VMEM((_CV,), jnp.float32),
    ],
)(_val_sc_body)


def _assoc_update_body(a2_ref, a1_ref, ma_ref, vp_ref, mv_ref, out_ref):
    scale = (jnp.float32(1.0) / jnp.float32(_S)) * jnp.float32(_LR)
    upd = jax.lax.dot_general(a2_ref[...], a1_ref[...],
                              (((0,), (0,)), ((), ())),
                              preferred_element_type=jnp.float32)
    ones = jnp.ones((_NW, 1), jnp.float32)
    val = mv_ref[...] + jax.lax.dot_general(
        vp_ref[...], ones, (((0,), (0,)), ((), ())),
        precision=jax.lax.Precision.HIGHEST,
        preferred_element_type=jnp.float32)
    out_ref[...] = jnp.concatenate([val, ma_ref[...] + upd * scale], axis=1)


def kernel(keys, targets, proj_value, clique_encoder, proj_assoc,
           mem_value_val, mem_value_assoc):
    nb = _B // _BM
    a1, a1f, dbc = pl.pallas_call(
        _encode_body,
        grid=(nb,),
        in_specs=[
            pl.BlockSpec((_BM, _D), lambda i: (i, 0)),
            pl.BlockSpec((_CV, _D), lambda i: (0, 0)),
            pl.BlockSpec((_CV, 1), lambda i: (0, 0)),
            pl.BlockSpec((_BM, 1), lambda i: (i, 0)),
        ],
        out_specs=[
            pl.BlockSpec((_BM, _CV), lambda i: (i, 0)),
            pl.BlockSpec((_BM, _CV), lambda i: (i, 0)),
            pl.BlockSpec((_BM, _L), lambda i: (i, 0)),
        ],
        out_shape=[
            jax.ShapeDtypeStruct((_B, _CV), jnp.bfloat16),
            jax.ShapeDtypeStruct((_B, _CV), jnp.float32),
            jax.ShapeDtypeStruct((_B, _L), jnp.float32),
        ],
    )(keys, proj_value, mem_value_val, targets)

    val_partials = _val_sc(a1f, dbc)

    a2 = pl.pallas_call(
        _assoc_encode_body,
        grid=(nb,),
        in_specs=[
            pl.BlockSpec((_BM, _CV), lambda i: (i, 0)),
            pl.BlockSpec((_CV, _CA), lambda i: (0, 0)),
            pl.BlockSpec((_CA, _CA), lambda i: (0, 0)),
        ],
        out_specs=pl.BlockSpec((_BM, _CA), lambda i: (i, 0)),
        out_shape=jax.ShapeDtypeStruct((_B, _CA), jnp.bfloat16),
    )(a1, clique_encoder, proj_assoc)

    bn = 256
    out = pl.pallas_call(
        _assoc_update_body,
        grid=(_CA // bn,),
        in_specs=[
            pl.BlockSpec((_B, bn), lambda j: (0, j)),
            pl.BlockSpec((_B, _CV), lambda j: (0, 0)),
            pl.BlockSpec((bn, _CV), lambda j: (j, 0)),
            pl.BlockSpec((_NW, bn), lambda j: (0, j)),
            pl.BlockSpec((bn, 1), lambda j: (j, 0)),
        ],
        out_specs=pl.BlockSpec((bn, 1 + _CV), lambda j: (j, 0)),
        out_shape=jax.ShapeDtypeStruct((_CA, 1 + _CV), jnp.float32),
    )(a2, a1, mem_value_assoc, val_partials, mem_value_val)

    return out
